# baseline (speedup 1.0000x reference)
import functools

import jax
import jax.numpy as jnp
from jax.experimental import pallas as pl
from jax.experimental.pallas import tpu as pltpu

F32 = jnp.float32
GTU_KS = (3, 5, 7)

B = 8
N = 8
BN = B * N
CIN = 2
C = 32
SC = 64
SEQ = 12
T0 = 19
NLAYERS = 3
K = 3
OUT_DIM = 12
EPS = 1e-5

TS = (19, 13, 7)


def _dot(a, b):
    return jnp.dot(a, b, preferred_element_type=F32)


def _dot_bt(a, b):
    return jax.lax.dot_general(a, b, (((1,), (1,)), ((), ())),
                               preferred_element_type=F32)


def _dot_tb(a, b):
    return jax.lax.dot_general(a, b, (((0,), (0,)), ((), ())),
                               preferred_element_type=F32)


def _softmax0(x):
    m = jnp.max(x, axis=0, keepdims=True)
    e = jnp.exp(x - m)
    return e / jnp.sum(e, axis=0, keepdims=True)


def _iota(shape, d):
    return jax.lax.broadcasted_iota(jnp.int32, shape, d)


def _eqf(a, b):
    return (a == b).astype(F32)


_MROWS = []
for _i in range(NLAYERS):
    _t = TS[_i]
    for _k in GTU_KS:
        _MROWS.append((_t, _t - _k + 1))
    for _k in GTU_KS:
        _MROWS.append((_t - 6, _t - _k + 1))


def _extract_body(*refs):
    n = len(_MROWS)
    for m, (_, to) in enumerate(_MROWS):
        row = refs[m][0:1, :]
        sel = _eqf(_iota((to * BN, to), 0), _iota((to * BN, to), 1) * BN)
        refs[n + m][0] = _dot(row, sel)


def _extract_factors(mats):
    in_specs = []
    out_specs = []
    out_shapes = []
    for mat, (tr, to) in zip(mats, _MROWS):
        in_specs.append(pl.BlockSpec(
            (8, mat.shape[1]),
            lambda t, _r=tr: (jnp.minimum(t, _r - 1) * 8, 0)))
        out_shapes.append(jax.ShapeDtypeStruct((tr, 1, to), F32))
        out_specs.append(pl.BlockSpec(
            (1, 1, to), lambda t, _r=tr: (jnp.minimum(t, _r - 1), 0, 0)))
    return pl.pallas_call(
        _extract_body,
        out_shape=tuple(out_shapes),
        grid=(T0,),
        in_specs=in_specs,
        out_specs=tuple(out_specs),
        compiler_params=pltpu.CompilerParams(
            dimension_semantics=("arbitrary",)),
    )(*mats)


def _gtu_bank(X, T, wpq, bp, bq):
    outs = []
    tap = 0
    for j, k in enumerate(GTU_KS):
        rows = (T - k + 1) * N
        acc = jnp.zeros((rows, 2 * C), F32)
        for dt in range(k):
            acc = acc + _dot(X[dt * N: dt * N + rows, :],
                             wpq[(tap + dt) * C:(tap + dt + 1) * C, :])
        tap += k
        p = acc[:, :C] + bp[:, j * C:(j + 1) * C]
        q = acc[:, C:] + bq[:, j * C:(j + 1) * C]
        outs.append(jnp.tanh(p) * jax.nn.sigmoid(q))
    return outs


def _kron_apply(a_ref, to, g):
    a = a_ref[:, 0, :]
    t = a.shape[0]
    ut = _eqf(_iota((t * N, t), 0) // N, _iota((t * N, t), 1))
    uto = _eqf(_iota((to * N, to), 0) // N, _iota((to * N, to), 1))
    big = _dot_bt(_dot(ut, a), uto)
    big = big * _eqf(_iota((t * N, to * N), 0) % N,
                     _iota((t * N, to * N), 1) % N)
    return _dot(big, g)


def _bias_col(bf_ref, t_len, b):
    parts = [bf_ref[pl.ds(t * BN + b * N, N), :] for t in range(t_len)]
    return jnp.concatenate(parts, axis=0)


def _row_of_col0(m_ref, sel):
    return jax.lax.dot_general(m_ref[:, 0:1], sel, (((0,), (1,)), ((), ())),
                               preferred_element_type=F32)


def _body(idx, nin, *refs):
    o_ref = refs[nin]
    b = pl.program_id(0)

    def R(name):
        return refs[idx[name]]

    wpq = jnp.concatenate([R("gtu_wp")[...], R("gtu_wq")[...]], axis=1)
    bp = R("gtu_bp")[...]
    bq = R("gtu_bq")[...]
    cheb = R("cheb")[...]
    theta = R("theta")[...]

    xs = R("x2")[0]
    xb = _dot_tb(xs, R("start_w")[...])
    sh = (T0 * N, N * SEQ)
    perm = (_eqf(_iota(sh, 0) % N, _iota(sh, 1) // SEQ)
            * _eqf(_iota(sh, 0) // N, _iota(sh, 1) % SEQ + (T0 - SEQ)))
    x = _dot(perm, xb) + R("start_b")[...]

    s8 = (N, N * SEQ)
    smask = _eqf(_iota(s8, 0), _iota(s8, 1) // SEQ)
    rmat = _eqf(_iota((N * SEQ, SEQ), 0) % SEQ, _iota((N * SEQ, SEQ), 1))
    skip_last = R("skip0_b")[...]
    for cin in range(CIN):
        a_c = _dot(jnp.broadcast_to(xs[cin:cin + 1, :], s8) * smask, rmat)
        wsel = _eqf(_iota((SEQ, T0 * CIN), 0) * CIN + (T0 - SEQ) * CIN + cin,
                    _iota((SEQ, T0 * CIN), 1))
        skip_last = skip_last + _dot(a_c, _dot(wsel, R("skip0_w")[...]))

    T = T0
    for i in range(NLAYERS):
        T_out = T - 6
        kw = T_out
        to_s = T - kw + 1
        residual = x

        g = _gtu_bank(x, T, wpq, bp, bq)
        tc = _bias_col(R("l%d_bF1" % i), T, b)
        for j, k in enumerate(GTU_KS):
            tc = tc + _kron_apply(R("l%d_A1_%d" % (i, j)), T - k + 1, g[j])
        x_new = jnp.maximum(x + tc, 0.0)

        s = jnp.zeros((N, SC), F32) + R("l%d_skip_b" % i)[...]
        skw = R("l%d_skip_w" % i)[...]
        for dt in range(kw):
            r0 = (to_s - 1 + dt) * N
            s = s + _dot(x_new[r0:r0 + N, :], skw[dt * C:(dt + 1) * C, :])
        skip_last = skip_last + s

        r8t = R("l%d_R8T" % i)[...]
        rnl = R("l%d_RNl" % i)[...]
        selm = _eqf(_iota((T * N, N * C), 1),
                    (_iota((T * N, N * C), 0) % N) * C)
        u1rep = _row_of_col0(R("l%d_U1k" % i), selm)
        selc = _eqf(_iota((C, N * C), 1), _iota((C, N * C), 0))
        u3row = _row_of_col0(R("l%d_U3k" % i), selc)
        w3row = _row_of_col0(R("l%d_W3k" % i), selc)

        lhs1 = _dot(_dot(r8t * u1rep, x_new), R("l%d_U2" % i)[...])
        v3 = _dot_bt(u3row, x_new)
        rhs1t = _dot(r8t * v3, rnl)
        prod1 = _dot_bt(lhs1, rhs1t)
        e = _dot(R("l%d_Ve" % i)[...],
                 jax.nn.sigmoid(prod1 + R("l%d_be" % i)[...]))
        t_att = _softmax0(e)

        w1t = _dot(R("l%d_W1row" % i)[...], t_att)
        k1 = R("l%d_Krep" % i)[...] * _dot(w1t, r8t)
        r = _dot(k1, x_new)
        lhs2 = _dot(r, R("l%d_W2" % i)[...])
        vw3 = _dot_bt(w3row, x_new)
        rhs2 = _dot(t_att, _dot(r8t * vw3, rnl))
        prod2 = _dot(lhs2, rhs2)
        s_f = _dot(R("l%d_Vs" % i)[...],
                   jax.nn.sigmoid(prod2 + R("l%d_bs" % i)[...]))
        s_att = _softmax0(s_f)

        mask = R("l%d_MaskTT" % i)[...]
        acc = jnp.zeros((T * N, C), F32)
        for kk in range(K):
            a = cheb[kk * N:(kk + 1) * N, :] * s_att
            big = _dot_bt(_dot_bt(rnl, a), rnl) * mask
            acc = acc + _dot(_dot(big, x_new), theta[kk * C:(kk + 1) * C, :])
        xg = jnp.maximum(acc, 0.0)

        g = _gtu_bank(xg, T, wpq, bp, bq)
        tc2 = _bias_col(R("l%d_bF2" % i), T_out, b)
        for j, k in enumerate(GTU_KS):
            tc2 = tc2 + _kron_apply(R("l%d_A2_%d" % (i, j)), T - k + 1, g[j])
        off = (T - T_out) * N
        xn2 = jnp.maximum(xg[off:, :] + tc2, 0.0) + residual[off:, :]
        cnt = float(T_out * N * C)
        mu = jnp.sum(xn2) / cnt
        d = xn2 - mu
        var = jnp.sum(d * d) / cnt
        x = d * jax.lax.rsqrt(var + EPS)
        T = T_out

    sk = _dot(x, R("skipE_w")[...]) + R("skipE_b")[...] + skip_last
    h = jnp.maximum(sk, 0.0)
    h = jnp.maximum(_dot(h, R("end1_w")[...]) + R("end1_b")[...], 0.0)
    res = _dot(h, R("end2_w")[...]) + R("end2_b")[...]
    eye = _eqf(_iota((N, N), 0), _iota((N, N), 1))
    o_ref[0] = _dot_tb(res, eye)


def kernel(x_in, start_w, skip0_w, gtu_wp, gtu_wq, theta, cheb, start_b, skip0_b, gtu_bp, gtu_bq, l0_skip_w, l0_skip_b, l0_M1_0, l0_M1_1, l0_M1_2, l0_M2_0, l0_M2_1, l0_M2_2, l0_bF1, l0_bF2, l0_U1k, l0_U2, l0_U3k, l0_Ve, l0_be, l0_W1row, l0_W2, l0_W3k, l0_Vs, l0_bs, l0_R8T, l0_Krep, l0_RNl, l0_MaskTT, l1_skip_w, l1_skip_b, l1_M1_0, l1_M1_1, l1_M1_2, l1_M2_0, l1_M2_1, l1_M2_2, l1_bF1, l1_bF2, l1_U1k, l1_U2, l1_U3k, l1_Ve, l1_be, l1_W1row, l1_W2, l1_W3k, l1_Vs, l1_bs, l1_R8T, l1_Krep, l1_RNl, l1_MaskTT, l2_skip_w, l2_skip_b, l2_M1_0, l2_M1_1, l2_M1_2, l2_M2_0, l2_M2_1, l2_M2_2, l2_bF1, l2_bF2, l2_U1k, l2_U2, l2_U3k, l2_Ve, l2_be, l2_W1row, l2_W2, l2_W3k, l2_Vs, l2_bs, l2_R8T, l2_Krep, l2_RNl, l2_MaskTT, skipE_w, end1_w, end2_w, skipE_b, end1_b, end2_b):
    lraw = [
        dict(skip_w=l0_skip_w, skip_b=l0_skip_b,
             M1=(l0_M1_0, l0_M1_1, l0_M1_2), M2=(l0_M2_0, l0_M2_1, l0_M2_2),
             bF1=l0_bF1, bF2=l0_bF2, U1k=l0_U1k, U2=l0_U2, U3k=l0_U3k,
             Ve=l0_Ve, be=l0_be, W1row=l0_W1row, W2=l0_W2, W3k=l0_W3k,
             Vs=l0_Vs, bs=l0_bs, R8T=l0_R8T, Krep=l0_Krep, RNl=l0_RNl,
             MaskTT=l0_MaskTT),
        dict(skip_w=l1_skip_w, skip_b=l1_skip_b,
             M1=(l1_M1_0, l1_M1_1, l1_M1_2), M2=(l1_M2_0, l1_M2_1, l1_M2_2),
             bF1=l1_bF1, bF2=l1_bF2, U1k=l1_U1k, U2=l1_U2, U3k=l1_U3k,
             Ve=l1_Ve, be=l1_be, W1row=l1_W1row, W2=l1_W2, W3k=l1_W3k,
             Vs=l1_Vs, bs=l1_bs, R8T=l1_R8T, Krep=l1_Krep, RNl=l1_RNl,
             MaskTT=l1_MaskTT),
        dict(skip_w=l2_skip_w, skip_b=l2_skip_b,
             M1=(l2_M1_0, l2_M1_1, l2_M1_2), M2=(l2_M2_0, l2_M2_1, l2_M2_2),
             bF1=l2_bF1, bF2=l2_bF2, U1k=l2_U1k, U2=l2_U2, U3k=l2_U3k,
             Ve=l2_Ve, be=l2_be, W1row=l2_W1row, W2=l2_W2, W3k=l2_W3k,
             Vs=l2_Vs, bs=l2_bs, R8T=l2_R8T, Krep=l2_Krep, RNl=l2_RNl,
             MaskTT=l2_MaskTT),
    ]

    mats = []
    for lp in lraw:
        mats.extend(lp["M1"])
        mats.extend(lp["M2"])
    factors = _extract_factors(mats)

    x2 = x_in.reshape(B, CIN, N * SEQ)

    args = []
    specs = []
    idx = {}

    def add(name, arr, per_sample=False):
        idx[name] = len(args)
        args.append(arr)
        if per_sample:
            blk = (1,) + arr.shape[1:]
            specs.append(pl.BlockSpec(blk, lambda i: (i,) + (0,) * (arr.ndim - 1)))
        else:
            nd = arr.ndim
            specs.append(pl.BlockSpec(arr.shape, lambda i, _n=nd: (0,) * _n))

    add("x2", x2, per_sample=True)
    for nm, arr in (("start_w", start_w), ("start_b", start_b),
                    ("skip0_w", skip0_w), ("skip0_b", skip0_b),
                    ("gtu_wp", gtu_wp), ("gtu_wq", gtu_wq),
                    ("gtu_bp", gtu_bp), ("gtu_bq", gtu_bq),
                    ("cheb", cheb), ("theta", theta),
                    ("skipE_w", skipE_w), ("skipE_b", skipE_b),
                    ("end1_w", end1_w), ("end1_b", end1_b),
                    ("end2_w", end2_w), ("end2_b", end2_b)):
        add(nm, arr)

    fpos = 0
    for i, lp in enumerate(lraw):
        for j in range(3):
            add("l%d_A1_%d" % (i, j), factors[fpos + j])
        for j in range(3):
            add("l%d_A2_%d" % (i, j), factors[fpos + 3 + j])
        fpos += 6
        add("l%d_bF1" % i, lp["bF1"])
        add("l%d_bF2" % i, lp["bF2"])
        for nm in ("skip_w", "skip_b", "U1k", "U2", "U3k", "Ve", "be",
                   "W1row", "W2", "W3k", "Vs", "bs", "R8T", "Krep", "RNl",
                   "MaskTT"):
            add("l%d_%s" % (i, nm), lp[nm])

    nin = len(args)
    out = pl.pallas_call(
        functools.partial(_body, idx, nin),
        out_shape=jax.ShapeDtypeStruct((B, OUT_DIM, N), F32),
        grid=(B,),
        in_specs=specs,
        out_specs=pl.BlockSpec((1, OUT_DIM, N), lambda i: (i, 0, 0)),
        compiler_params=pltpu.CompilerParams(dimension_semantics=("parallel",)),
    )(*args)
    return out[..., None]
